# baseline (speedup 1.0000x reference)
import functools

import jax
import jax.numpy as jnp
from jax import lax
from jax.experimental import pallas as pl
from jax.experimental.pallas import tpu as pltpu
from jax.experimental.pallas import tpu_sc as plsc

N = 10000
D = 128
NG = 64

NC = 2
NS = 16
NW = NC * NS
CHUNK = 128
CA = 48
CB = 112
NSH = 10112
RZ = NSH // NS


def _unpack_chunk(packed_v, j, srcbuf, dstbuf, b):
    for i in range(CHUNK // 16):
        pk = packed_v[j, pl.ds(i * 16, 16)]
        srcbuf[b, pl.ds(i * 16, 16)] = pk & 0xFFFF
        dstbuf[b, pl.ds(i * 16, 16)] = lax.shift_right_logical(pk, 16)


def _sc_agg_body(x_hbm, packed_hbm, zero_hbm, out_hbm,
                 packed_v, srcbuf, dstbuf, rows_v, shared, sem):
    cid = lax.axis_index("c")
    sid = lax.axis_index("s")
    pltpu.sync_copy(zero_hbm.at[pl.ds(sid * RZ, RZ)],
                    shared.at[pl.ds(sid * RZ, RZ)])
    nj = lax.select(cid == 0, CA, CB)
    off = lax.select(cid == 0, sid * CA, NS * CA + sid * CB)

    @pl.when(cid == 0)
    def _():
        pltpu.sync_copy(packed_hbm.at[pl.ds(off, CA)],
                        packed_v.at[pl.ds(0, CA)])

    @pl.when(cid != 0)
    def _():
        pltpu.sync_copy(packed_hbm.at[pl.ds(off, CB)],
                        packed_v.at[pl.ds(0, CB)])

    plsc.subcore_barrier()

    _unpack_chunk(packed_v, 0, srcbuf, dstbuf, 0)
    pltpu.async_copy(x_hbm.at[srcbuf.at[0]], rows_v.at[0], sem)

    def step(j, carry):
        b = lax.rem(j, 2)
        jn = lax.min(j + 1, nj - 1)
        _unpack_chunk(packed_v, jn, srcbuf, dstbuf, 1 - b)
        pltpu.make_async_copy(x_hbm.at[srcbuf.at[b]], rows_v.at[b],
                              sem).wait()

        @pl.when(j + 1 < nj)
        def _():
            pltpu.async_copy(x_hbm.at[srcbuf.at[1 - b]], rows_v.at[1 - b],
                             sem)

        pltpu.sync_copy(rows_v.at[b], shared.at[dstbuf.at[b]], add=True)
        return carry

    lax.fori_loop(0, nj, step, 0)
    plsc.subcore_barrier()
    pltpu.sync_copy(shared.at[pl.ds(sid * RZ, RZ)],
                    out_hbm.at[cid, pl.ds(sid * RZ, RZ)])


def _sc_agg(x, packed, zinit):
    mesh = plsc.VectorSubcoreMesh(core_axis_name="c", subcore_axis_name="s")
    return pl.kernel(
        _sc_agg_body,
        out_type=jax.ShapeDtypeStruct((NC, NSH, D), jnp.float32),
        mesh=mesh,
        scratch_types=[
            pltpu.VMEM((max(CA, CB), CHUNK), jnp.int32),
            pltpu.VMEM((2, CHUNK), jnp.int32),
            pltpu.VMEM((2, CHUNK), jnp.int32),
            pltpu.VMEM((2, CHUNK, D), jnp.float32),
            pltpu.VMEM_SHARED((NSH, D), jnp.float32),
            pltpu.SemaphoreType.DMA,
        ],
    )(x, packed, zinit)


def _layer_body(x_ref, p_ref, w1_ref, b1_ref, w2_ref, b2_ref, g_ref, be_ref,
                o_ref):
    h = x_ref[...] + p_ref[0, :N, :] + p_ref[1, :N, :]
    a = jnp.maximum(
        jnp.dot(h, w1_ref[...], preferred_element_type=jnp.float32)
        + b1_ref[...], 0.0)
    z = (jnp.dot(a, w2_ref[...], preferred_element_type=jnp.float32)
         + b2_ref[...])
    m = jnp.mean(z, axis=0, keepdims=True)
    zc = z - m
    v = jnp.mean(zc * zc, axis=0, keepdims=True)
    o_ref[...] = jnp.maximum(
        zc * lax.rsqrt(v + 1e-5) * g_ref[...] + be_ref[...], 0.0)


def _tc_layer(x, p, w1, b1, w2, b2, g, be):
    return pl.pallas_call(
        _layer_body,
        out_shape=jax.ShapeDtypeStruct((N, D), jnp.float32),
        compiler_params=pltpu.CompilerParams(
            vmem_limit_bytes=100 * 1024 * 1024),
    )(x, p, w1, b1.reshape(1, -1), w2, b2.reshape(1, -1),
      g.reshape(1, -1), be.reshape(1, -1))


def _head_body(h1_ref, h2_ref, h3_ref, bt_ref, wl_ref, bl_ref, o_ref):
    bt = bt_ref[...]
    gi = lax.broadcasted_iota(jnp.int32, (NG, N), 0)
    oneh = jnp.where(bt == gi, 1.0, 0.0)
    cnt = jnp.sum(oneh, axis=1, keepdims=True)
    inv = 1.0 / jnp.maximum(cnt, 1.0)
    p1 = jnp.dot(oneh, h1_ref[...], preferred_element_type=jnp.float32) * inv
    p2 = jnp.dot(oneh, h2_ref[...], preferred_element_type=jnp.float32) * inv
    p3 = jnp.dot(oneh, h3_ref[...], preferred_element_type=jnp.float32) * inv
    hf = jnp.concatenate([p1, p2, p3], axis=1)
    out = (jnp.dot(hf, wl_ref[...], preferred_element_type=jnp.float32)
           + bl_ref[...])
    nrm = jnp.sqrt(jnp.sum(out * out, axis=1, keepdims=True))
    o_ref[...] = out / jnp.maximum(nrm, 1e-12)


def _tc_head(h1, h2, h3, batch, wl, bl):
    return pl.pallas_call(
        _head_body,
        out_shape=jax.ShapeDtypeStruct((NG, wl.shape[1]), jnp.float32),
        compiler_params=pltpu.CompilerParams(
            vmem_limit_bytes=100 * 1024 * 1024),
    )(h1, h2, h3, batch.reshape(1, N), wl, bl.reshape(1, -1))


def kernel(x, edge_index, batch, W1_1, b1_1, W2_1, b2_1, g_1, be_1,
           W1_2, b1_2, W2_2, b2_2, g_2, be_2, W1_3, b1_3, W2_3, b2_3,
           g_3, be_3, Wlin, blin):
    E = edge_index.shape[1]
    nchunks = NS * (CA + CB)
    pad = nchunks * CHUNK - E
    src = jnp.concatenate([edge_index[0], jnp.zeros((pad,), jnp.int32)])
    dst = jnp.concatenate([edge_index[1],
                           jnp.full((pad,), N, jnp.int32)])
    packed = (src | (dst << 16)).reshape(nchunks, CHUNK)
    zinit = jnp.zeros((NSH, D), jnp.float32)

    p = _sc_agg(x, packed, zinit)
    h1 = _tc_layer(x, p, W1_1, b1_1, W2_1, b2_1, g_1, be_1)
    p = _sc_agg(h1, packed, zinit)
    h2 = _tc_layer(h1, p, W1_2, b1_2, W2_2, b2_2, g_2, be_2)
    p = _sc_agg(h2, packed, zinit)
    h3 = _tc_layer(h2, p, W1_3, b1_3, W2_3, b2_3, g_3, be_3)
    return _tc_head(h1, h2, h3, batch, Wlin, blin)


# speedup vs baseline: 1.0618x; 1.0618x over previous
import functools

import jax
import jax.numpy as jnp
from jax import lax
from jax.experimental import pallas as pl
from jax.experimental.pallas import tpu as pltpu
from jax.experimental.pallas import tpu_sc as plsc

N = 10000
D = 128
NG = 64

NC = 2
NS = 16
NW = NC * NS
CHUNK = 128
CA = 80
CB = 80
NSH = 10112
RZ = NSH // NS


def _unpack_chunk(packed_v, j, srcbuf, dstbuf, b):
    for i in range(CHUNK // 16):
        pk = packed_v[j, pl.ds(i * 16, 16)]
        srcbuf[b, pl.ds(i * 16, 16)] = pk & 0xFFFF
        dstbuf[b, pl.ds(i * 16, 16)] = lax.shift_right_logical(pk, 16)


def _sc_agg_body(x_hbm, packed_hbm, zero_hbm, out_hbm,
                 packed_v, srcbuf, dstbuf, rows_v, shared, sem):
    cid = lax.axis_index("c")
    sid = lax.axis_index("s")
    pltpu.sync_copy(zero_hbm.at[pl.ds(sid * RZ, RZ)],
                    shared.at[pl.ds(sid * RZ, RZ)])
    nj = CA
    wid = sid * NC + cid
    pltpu.sync_copy(packed_hbm.at[pl.ds(wid * CA, CA)], packed_v)
    plsc.subcore_barrier()

    _unpack_chunk(packed_v, 0, srcbuf, dstbuf, 0)
    pltpu.async_copy(x_hbm.at[srcbuf.at[0]], rows_v.at[0], sem)

    def step(j, carry):
        b = lax.rem(j, 2)
        jn = lax.min(j + 1, nj - 1)
        _unpack_chunk(packed_v, jn, srcbuf, dstbuf, 1 - b)
        pltpu.make_async_copy(x_hbm.at[srcbuf.at[b]], rows_v.at[b],
                              sem).wait()

        @pl.when(j + 1 < nj)
        def _():
            pltpu.async_copy(x_hbm.at[srcbuf.at[1 - b]], rows_v.at[1 - b],
                             sem)

        pltpu.sync_copy(rows_v.at[b], shared.at[dstbuf.at[b]], add=True)
        return carry

    lax.fori_loop(0, nj, step, 0)
    plsc.subcore_barrier()
    pltpu.sync_copy(shared.at[pl.ds(sid * RZ, RZ)],
                    out_hbm.at[cid, pl.ds(sid * RZ, RZ)])


def _sc_agg(x, packed, zinit):
    mesh = plsc.VectorSubcoreMesh(core_axis_name="c", subcore_axis_name="s")
    return pl.kernel(
        _sc_agg_body,
        out_type=jax.ShapeDtypeStruct((NC, NSH, D), jnp.float32),
        mesh=mesh,
        scratch_types=[
            pltpu.VMEM((max(CA, CB), CHUNK), jnp.int32),
            pltpu.VMEM((2, CHUNK), jnp.int32),
            pltpu.VMEM((2, CHUNK), jnp.int32),
            pltpu.VMEM((2, CHUNK, D), jnp.float32),
            pltpu.VMEM_SHARED((NSH, D), jnp.float32),
            pltpu.SemaphoreType.DMA,
        ],
    )(x, packed, zinit)


def _layer_body(x_ref, p_ref, w1_ref, b1_ref, w2_ref, b2_ref, g_ref, be_ref,
                o_ref):
    h = x_ref[...] + p_ref[0, :N, :] + p_ref[1, :N, :]
    a = jnp.maximum(
        jnp.dot(h, w1_ref[...], preferred_element_type=jnp.float32)
        + b1_ref[...], 0.0)
    z = (jnp.dot(a, w2_ref[...], preferred_element_type=jnp.float32)
         + b2_ref[...])
    m = jnp.mean(z, axis=0, keepdims=True)
    zc = z - m
    v = jnp.mean(zc * zc, axis=0, keepdims=True)
    o_ref[...] = jnp.maximum(
        zc * lax.rsqrt(v + 1e-5) * g_ref[...] + be_ref[...], 0.0)


def _tc_layer(x, p, w1, b1, w2, b2, g, be):
    return pl.pallas_call(
        _layer_body,
        out_shape=jax.ShapeDtypeStruct((N, D), jnp.float32),
        compiler_params=pltpu.CompilerParams(
            vmem_limit_bytes=100 * 1024 * 1024),
    )(x, p, w1, b1.reshape(1, -1), w2, b2.reshape(1, -1),
      g.reshape(1, -1), be.reshape(1, -1))


def _head_body(h1_ref, h2_ref, h3_ref, bt_ref, wl_ref, bl_ref, o_ref):
    bt = bt_ref[...]
    gi = lax.broadcasted_iota(jnp.int32, (NG, N), 0)
    oneh = jnp.where(bt == gi, 1.0, 0.0)
    cnt = jnp.sum(oneh, axis=1, keepdims=True)
    inv = 1.0 / jnp.maximum(cnt, 1.0)
    p1 = jnp.dot(oneh, h1_ref[...], preferred_element_type=jnp.float32) * inv
    p2 = jnp.dot(oneh, h2_ref[...], preferred_element_type=jnp.float32) * inv
    p3 = jnp.dot(oneh, h3_ref[...], preferred_element_type=jnp.float32) * inv
    hf = jnp.concatenate([p1, p2, p3], axis=1)
    out = (jnp.dot(hf, wl_ref[...], preferred_element_type=jnp.float32)
           + bl_ref[...])
    nrm = jnp.sqrt(jnp.sum(out * out, axis=1, keepdims=True))
    o_ref[...] = out / jnp.maximum(nrm, 1e-12)


def _tc_head(h1, h2, h3, batch, wl, bl):
    return pl.pallas_call(
        _head_body,
        out_shape=jax.ShapeDtypeStruct((NG, wl.shape[1]), jnp.float32),
        compiler_params=pltpu.CompilerParams(
            vmem_limit_bytes=100 * 1024 * 1024),
    )(h1, h2, h3, batch.reshape(1, N), wl, bl.reshape(1, -1))


def kernel(x, edge_index, batch, W1_1, b1_1, W2_1, b2_1, g_1, be_1,
           W1_2, b1_2, W2_2, b2_2, g_2, be_2, W1_3, b1_3, W2_3, b2_3,
           g_3, be_3, Wlin, blin):
    E = edge_index.shape[1]
    nchunks = NS * (CA + CB)
    pad = nchunks * CHUNK - E
    src = jnp.concatenate([edge_index[0], jnp.zeros((pad,), jnp.int32)])
    dst = jnp.concatenate([edge_index[1],
                           jnp.full((pad,), N, jnp.int32)])
    packed = (src | (dst << 16)).reshape(nchunks, CHUNK)
    zinit = jnp.zeros((NSH, D), jnp.float32)

    p = _sc_agg(x, packed, zinit)
    h1 = _tc_layer(x, p, W1_1, b1_1, W2_1, b2_1, g_1, be_1)
    p = _sc_agg(h1, packed, zinit)
    h2 = _tc_layer(h1, p, W1_2, b1_2, W2_2, b2_2, g_2, be_2)
    p = _sc_agg(h2, packed, zinit)
    h3 = _tc_layer(h2, p, W1_3, b1_3, W2_3, b2_3, g_3, be_3)
    return _tc_head(h1, h2, h3, batch, Wlin, blin)


# speedup vs baseline: 3.6973x; 3.4822x over previous
import functools

import jax
import jax.numpy as jnp
from jax import lax
from jax.experimental import pallas as pl
from jax.experimental.pallas import tpu as pltpu
from jax.experimental.pallas import tpu_sc as plsc

N = 10000
D = 128
NG = 64

NC = 2
NS = 16
NW = NC * NS
CHUNK = 128
NSH = 10112
RZ = NSH // NS


def _unpack_chunk(packed_v, j, srcbuf, dstbuf, b):
    for i in range(CHUNK // 16):
        pk = packed_v[j, pl.ds(i * 16, 16)]
        srcbuf[b, pl.ds(i * 16, 16)] = pk & 0xFFFF
        dstbuf[b, pl.ds(i * 16, 16)] = lax.shift_right_logical(pk, 16)


def _sc_agg_body(x_hbm, packed_hbm, zero_hbm, out_hbm,
                 packed_v, srcbuf, dstbuf, rows_v, shared, sem):
    cid = lax.axis_index("c")
    sid = lax.axis_index("s")
    pltpu.sync_copy(zero_hbm.at[pl.ds(sid * RZ, RZ)],
                    shared.at[pl.ds(sid * RZ, RZ)])
    nj = packed_hbm.shape[1]
    wid = sid * NC + cid
    pltpu.sync_copy(packed_hbm.at[wid], packed_v)
    plsc.subcore_barrier()

    _unpack_chunk(packed_v, 0, srcbuf, dstbuf, 0)
    pltpu.async_copy(x_hbm.at[srcbuf.at[0]], rows_v.at[0], sem)

    def step(j, carry):
        b = lax.rem(j, 2)
        jn = lax.min(j + 1, nj - 1)
        _unpack_chunk(packed_v, jn, srcbuf, dstbuf, 1 - b)
        pltpu.make_async_copy(x_hbm.at[srcbuf.at[b]], rows_v.at[b],
                              sem).wait()

        @pl.when(j + 1 < nj)
        def _():
            pltpu.async_copy(x_hbm.at[srcbuf.at[1 - b]], rows_v.at[1 - b],
                             sem)

        pltpu.sync_copy(rows_v.at[b], shared.at[dstbuf.at[b]], add=True)
        return carry

    lax.fori_loop(0, nj, step, 0)
    plsc.subcore_barrier()
    pltpu.sync_copy(shared.at[pl.ds(sid * RZ, RZ)],
                    out_hbm.at[cid, pl.ds(sid * RZ, RZ)])


def _sc_agg(x, packed, zinit):
    mesh = plsc.VectorSubcoreMesh(core_axis_name="c", subcore_axis_name="s")
    return pl.kernel(
        _sc_agg_body,
        out_type=jax.ShapeDtypeStruct((NC, NSH, D), jnp.float32),
        mesh=mesh,
        scratch_types=[
            pltpu.VMEM((packed.shape[1], CHUNK), jnp.int32),
            pltpu.VMEM((2, CHUNK), jnp.int32),
            pltpu.VMEM((2, CHUNK), jnp.int32),
            pltpu.VMEM((2, CHUNK, D), jnp.float32),
            pltpu.VMEM_SHARED((NSH, D), jnp.float32),
            pltpu.SemaphoreType.DMA,
        ],
    )(x, packed, zinit)


def _layer_body(x_ref, p_ref, w1_ref, b1_ref, w2_ref, b2_ref, g_ref, be_ref,
                o_ref):
    h = x_ref[...] + p_ref[0, :N, :] + p_ref[1, :N, :]
    a = jnp.maximum(
        jnp.dot(h, w1_ref[...], preferred_element_type=jnp.float32)
        + b1_ref[...], 0.0)
    z = (jnp.dot(a, w2_ref[...], preferred_element_type=jnp.float32)
         + b2_ref[...])
    m = jnp.mean(z, axis=0, keepdims=True)
    zc = z - m
    v = jnp.mean(zc * zc, axis=0, keepdims=True)
    o_ref[...] = jnp.maximum(
        zc * lax.rsqrt(v + 1e-5) * g_ref[...] + be_ref[...], 0.0)


def _tc_layer(x, p, w1, b1, w2, b2, g, be):
    return pl.pallas_call(
        _layer_body,
        out_shape=jax.ShapeDtypeStruct((N, D), jnp.float32),
        compiler_params=pltpu.CompilerParams(
            vmem_limit_bytes=100 * 1024 * 1024),
    )(x, p, w1, b1.reshape(1, -1), w2, b2.reshape(1, -1),
      g.reshape(1, -1), be.reshape(1, -1))


def _head_body(h1_ref, h2_ref, h3_ref, bt_ref, wl_ref, bl_ref, o_ref):
    bt = bt_ref[...]
    gi = lax.broadcasted_iota(jnp.int32, (NG, N), 0)
    oneh = jnp.where(bt == gi, 1.0, 0.0)
    cnt = jnp.sum(oneh, axis=1, keepdims=True)
    inv = 1.0 / jnp.maximum(cnt, 1.0)
    p1 = jnp.dot(oneh, h1_ref[...], preferred_element_type=jnp.float32) * inv
    p2 = jnp.dot(oneh, h2_ref[...], preferred_element_type=jnp.float32) * inv
    p3 = jnp.dot(oneh, h3_ref[...], preferred_element_type=jnp.float32) * inv
    hf = jnp.concatenate([p1, p2, p3], axis=1)
    out = (jnp.dot(hf, wl_ref[...], preferred_element_type=jnp.float32)
           + bl_ref[...])
    nrm = jnp.sqrt(jnp.sum(out * out, axis=1, keepdims=True))
    o_ref[...] = out / jnp.maximum(nrm, 1e-12)


def _tc_head(h1, h2, h3, batch, wl, bl):
    return pl.pallas_call(
        _head_body,
        out_shape=jax.ShapeDtypeStruct((NG, wl.shape[1]), jnp.float32),
        compiler_params=pltpu.CompilerParams(
            vmem_limit_bytes=100 * 1024 * 1024),
    )(h1, h2, h3, batch.reshape(1, N), wl, bl.reshape(1, -1))


def kernel(x, edge_index, batch, W1_1, b1_1, W2_1, b2_1, g_1, be_1,
           W1_2, b1_2, W2_2, b2_2, g_2, be_2, W1_3, b1_3, W2_3, b2_3,
           g_3, be_3, Wlin, blin):
    E = edge_index.shape[1]
    epw = -(-E // (NW * CHUNK)) * CHUNK
    pad = NW * epw - E
    ar = jnp.arange(pad, dtype=jnp.int32)
    src = jnp.concatenate([edge_index[0], ar % N])
    dst = jnp.concatenate([edge_index[1], N + ar % (NSH - N)])
    packed = (src | (dst << 16)).reshape(NW, epw // CHUNK, CHUNK)
    zinit = jnp.zeros((NSH, D), jnp.float32)

    p = _sc_agg(x, packed, zinit)
    h1 = _tc_layer(x, p, W1_1, b1_1, W2_1, b2_1, g_1, be_1)
    p = _sc_agg(h1, packed, zinit)
    h2 = _tc_layer(h1, p, W1_2, b1_2, W2_2, b2_2, g_2, be_2)
    p = _sc_agg(h2, packed, zinit)
    h3 = _tc_layer(h2, p, W1_3, b1_3, W2_3, b2_3, g_3, be_3)
    return _tc_head(h1, h2, h3, batch, Wlin, blin)


# speedup vs baseline: 4.5909x; 1.2417x over previous
import functools

import jax
import jax.numpy as jnp
from jax import lax
from jax.experimental import pallas as pl
from jax.experimental.pallas import tpu as pltpu
from jax.experimental.pallas import tpu_sc as plsc

N = 10000
D = 128
NG = 64

NC = 2
NS = 16
NW = NC * NS
CHUNK = 80
NBUF = 3
NSH = 10112
RZ = NSH // NS


def _unpack_chunk(packed_v, j, srcbuf, dstbuf, b):
    for i in range(CHUNK // 16):
        pk = packed_v[j, pl.ds(i * 16, 16)]
        srcbuf[b, pl.ds(i * 16, 16)] = pk & 0xFFFF
        dstbuf[b, pl.ds(i * 16, 16)] = lax.shift_right_logical(pk, 16)


def _sc_agg_body(x_hbm, packed_hbm, zero_hbm, out_hbm,
                 packed_v, srcbuf, dstbuf, rows_v, shared, sem):
    cid = lax.axis_index("c")
    sid = lax.axis_index("s")
    pltpu.sync_copy(zero_hbm.at[pl.ds(sid * RZ, RZ)],
                    shared.at[pl.ds(sid * RZ, RZ)])
    nj = packed_hbm.shape[1]
    wid = sid * NC + cid
    pltpu.sync_copy(packed_hbm.at[wid], packed_v)
    plsc.subcore_barrier()

    for b0 in range(NBUF - 1):
        _unpack_chunk(packed_v, b0, srcbuf, dstbuf, b0)
        pltpu.async_copy(x_hbm.at[srcbuf.at[b0]], rows_v.at[b0], sem)

    def step(j, carry):
        b = lax.rem(j, NBUF)
        b2 = lax.rem(j + NBUF - 1, NBUF)
        pltpu.make_async_copy(x_hbm.at[srcbuf.at[b]], rows_v.at[b],
                              sem).wait()

        @pl.when(j + NBUF - 1 < nj)
        def _():
            _unpack_chunk(packed_v, j + NBUF - 1, srcbuf, dstbuf, b2)
            pltpu.async_copy(x_hbm.at[srcbuf.at[b2]], rows_v.at[b2], sem)

        pltpu.sync_copy(rows_v.at[b], shared.at[dstbuf.at[b]], add=True)
        return carry

    lax.fori_loop(0, nj, step, 0)
    plsc.subcore_barrier()
    pltpu.sync_copy(shared.at[pl.ds(sid * RZ, RZ)],
                    out_hbm.at[cid, pl.ds(sid * RZ, RZ)])


def _sc_agg(x, packed, zinit):
    mesh = plsc.VectorSubcoreMesh(core_axis_name="c", subcore_axis_name="s")
    return pl.kernel(
        _sc_agg_body,
        out_type=jax.ShapeDtypeStruct((NC, NSH, D), jnp.float32),
        mesh=mesh,
        scratch_types=[
            pltpu.VMEM((packed.shape[1], CHUNK), jnp.int32),
            pltpu.VMEM((NBUF, CHUNK), jnp.int32),
            pltpu.VMEM((NBUF, CHUNK), jnp.int32),
            pltpu.VMEM((NBUF, CHUNK, D), jnp.float32),
            pltpu.VMEM_SHARED((NSH, D), jnp.float32),
            pltpu.SemaphoreType.DMA,
        ],
    )(x, packed, zinit)


def _layer_body(x_ref, p_ref, w1_ref, b1_ref, w2_ref, b2_ref, g_ref, be_ref,
                o_ref):
    h = x_ref[...] + p_ref[0, :N, :] + p_ref[1, :N, :]
    a = jnp.maximum(
        jnp.dot(h, w1_ref[...], preferred_element_type=jnp.float32)
        + b1_ref[...], 0.0)
    z = (jnp.dot(a, w2_ref[...], preferred_element_type=jnp.float32)
         + b2_ref[...])
    m = jnp.mean(z, axis=0, keepdims=True)
    zc = z - m
    v = jnp.mean(zc * zc, axis=0, keepdims=True)
    o_ref[...] = jnp.maximum(
        zc * lax.rsqrt(v + 1e-5) * g_ref[...] + be_ref[...], 0.0)


def _tc_layer(x, p, w1, b1, w2, b2, g, be):
    return pl.pallas_call(
        _layer_body,
        out_shape=jax.ShapeDtypeStruct((N, D), jnp.float32),
        compiler_params=pltpu.CompilerParams(
            vmem_limit_bytes=100 * 1024 * 1024),
    )(x, p, w1, b1.reshape(1, -1), w2, b2.reshape(1, -1),
      g.reshape(1, -1), be.reshape(1, -1))


def _pool_body(h_ref, bt_ref, o_ref):
    bt = bt_ref[...]
    gi = lax.broadcasted_iota(jnp.int32, (NG, N), 0)
    oneh = jnp.where(bt == gi, 1.0, 0.0)
    cnt = jnp.sum(oneh, axis=1, keepdims=True)
    inv = 1.0 / jnp.maximum(cnt, 1.0)
    o_ref[...] = (jnp.dot(oneh, h_ref[...],
                          preferred_element_type=jnp.float32) * inv)


def _tc_pool(h, batch):
    return pl.pallas_call(
        _pool_body,
        out_shape=jax.ShapeDtypeStruct((NG, D), jnp.float32),
        compiler_params=pltpu.CompilerParams(
            vmem_limit_bytes=100 * 1024 * 1024),
    )(h, batch.reshape(1, N))


def _head_body(p1_ref, p2_ref, p3_ref, wl_ref, bl_ref, o_ref):
    hf = jnp.concatenate([p1_ref[...], p2_ref[...], p3_ref[...]], axis=1)
    out = (jnp.dot(hf, wl_ref[...], preferred_element_type=jnp.float32)
           + bl_ref[...])
    nrm = jnp.sqrt(jnp.sum(out * out, axis=1, keepdims=True))
    o_ref[...] = out / jnp.maximum(nrm, 1e-12)


def _tc_head(p1, p2, p3, wl, bl):
    return pl.pallas_call(
        _head_body,
        out_shape=jax.ShapeDtypeStruct((NG, wl.shape[1]), jnp.float32),
        compiler_params=pltpu.CompilerParams(
            vmem_limit_bytes=100 * 1024 * 1024),
    )(p1, p2, p3, wl, bl.reshape(1, -1))


def kernel(x, edge_index, batch, W1_1, b1_1, W2_1, b2_1, g_1, be_1,
           W1_2, b1_2, W2_2, b2_2, g_2, be_2, W1_3, b1_3, W2_3, b2_3,
           g_3, be_3, Wlin, blin):
    E = edge_index.shape[1]
    epw = -(-E // (NW * CHUNK)) * CHUNK
    pad = NW * epw - E
    ar = jnp.arange(pad, dtype=jnp.int32)
    src = jnp.concatenate([edge_index[0], ar % N])
    dst = jnp.concatenate([edge_index[1], N + ar % (NSH - N)])
    packed = (src | (dst << 16)).reshape(NW, epw // CHUNK, CHUNK)
    zinit = jnp.zeros((NSH, D), jnp.float32)

    p = _sc_agg(x, packed, zinit)
    h1 = _tc_layer(x, p, W1_1, b1_1, W2_1, b2_1, g_1, be_1)
    p = _sc_agg(h1, packed, zinit)
    pool1 = _tc_pool(h1, batch)
    h2 = _tc_layer(h1, p, W1_2, b1_2, W2_2, b2_2, g_2, be_2)
    p = _sc_agg(h2, packed, zinit)
    pool2 = _tc_pool(h2, batch)
    h3 = _tc_layer(h2, p, W1_3, b1_3, W2_3, b2_3, g_3, be_3)
    pool3 = _tc_pool(h3, batch)
    return _tc_head(pool1, pool2, pool3, Wlin, blin)
